# baseline (speedup 1.0000x reference)
import functools

import jax
import jax.numpy as jnp
from jax import lax
from jax.experimental import pallas as pl
from jax.experimental.pallas import tpu as pltpu
from jax.experimental.pallas import tpu_sc as plsc


def kernel(batch, weight):
    B, = batch.shape
    V, D = weight.shape
    DP = 2 * D
    NC, NS = 2, 16
    NW = NC * NS
    assert B % NW == 0 and V % 2 == 0
    b_per_w = B // NW

    wpair = weight.reshape(V // 2, DP)

    mesh = plsc.VectorSubcoreMesh(core_axis_name="c", subcore_axis_name="s")

    @functools.partial(
        pl.kernel,
        mesh=mesh,
        out_type=jax.ShapeDtypeStruct((B, DP), jnp.float32),
        scratch_types=[
            pltpu.VMEM((b_per_w,), jnp.int32),
            pltpu.VMEM((b_per_w,), jnp.int32),
            pltpu.VMEM((b_per_w, DP), jnp.float32),
            pltpu.SemaphoreType.DMA,
        ],
    )
    def _emb(idx_hbm, table_hbm, out_hbm, idx_v, pair_v, rows_v, sem):
        wid = lax.axis_index("s") * NC + lax.axis_index("c")
        base = wid * b_per_w
        pltpu.sync_copy(idx_hbm.at[pl.ds(base, b_per_w)], idx_v)

        def halve(g, _):
            v = idx_v[pl.ds(g * 16, 16)]
            pair_v[pl.ds(g * 16, 16)] = lax.shift_right_logical(v, 1)
            return ()

        lax.fori_loop(0, b_per_w // 16, halve, (), unroll=8)

        pltpu.async_copy(table_hbm.at[pair_v], rows_v, sem).wait()

        def select(g, _):
            v16 = idx_v[pl.ds(g * 16, 16)]
            for l in range(16):
                off = (v16[l] & 1) * D
                row = rows_v.at[g * 16 + l]
                for v in range(D // 16):
                    row[pl.ds(v * 16, 16)] = row[pl.ds(off + v * 16, 16)]
            return ()

        lax.fori_loop(0, b_per_w // 16, select, ())
        pltpu.sync_copy(rows_v, out_hbm.at[pl.ds(base, b_per_w)])

    return _emb(batch, wpair)[:, :D]


# speedup vs baseline: 1.5885x; 1.5885x over previous
import functools

import jax
import jax.numpy as jnp
from jax import lax
from jax.experimental import pallas as pl
from jax.experimental.pallas import tpu as pltpu
from jax.experimental.pallas import tpu_sc as plsc


def kernel(batch, weight):
    B, = batch.shape
    V, D = weight.shape
    NC, NS = 2, 16
    NW = NC * NS
    assert B % NW == 0
    b_per_w = B // NW
    GB = 64
    n_b = b_per_w // GB

    mesh = plsc.VectorSubcoreMesh(core_axis_name="c", subcore_axis_name="s")

    @functools.partial(
        pl.kernel,
        mesh=mesh,
        out_type=jax.ShapeDtypeStruct((B, 2 * D), jnp.float32),
        scratch_types=[
            pltpu.VMEM((b_per_w,), jnp.int32),
            pltpu.VMEM((GB * 8, D), jnp.float32),
            pltpu.VMEM((GB, 2 * D), jnp.float32),
            pltpu.SemaphoreType.DMA,
        ],
    )
    def _emb(idx_hbm, table_hbm, out_hbm, idx_v, grp_v, rows_v, sem):
        wid = lax.axis_index("s") * NC + lax.axis_index("c")
        base = wid * b_per_w
        pltpu.sync_copy(idx_hbm.at[pl.ds(base, b_per_w)], idx_v)

        def batch_body(b, _):
            for q in range(GB // 16):
                v16 = idx_v[pl.ds(b * GB + q * 16, 16)]
                g16 = (v16 >> 3) << 3
                for l in range(16):
                    i8 = pl.multiple_of(g16[l], 8)
                    pltpu.async_copy(
                        table_hbm.at[pl.ds(i8, 8)],
                        grp_v.at[pl.ds((q * 16 + l) * 8, 8)],
                        sem,
                    )
            for k in range(GB):
                pltpu.make_async_copy(
                    table_hbm.at[pl.ds(0, 8)],
                    grp_v.at[pl.ds(k * 8, 8)],
                    sem,
                ).wait()
            for q in range(GB // 16):
                v16 = idx_v[pl.ds(b * GB + q * 16, 16)]
                for l in range(16):
                    src = grp_v.at[(q * 16 + l) * 8 + (v16[l] & 7)]
                    dst = rows_v.at[q * 16 + l]
                    for v in range(D // 16):
                        dst[pl.ds(v * 16, 16)] = src[pl.ds(v * 16, 16)]
            pltpu.sync_copy(rows_v, out_hbm.at[pl.ds(base + b * GB, GB)])
            return ()

        lax.fori_loop(0, n_b, batch_body, ())

    return _emb(batch, weight)[:, :D]
